# baseline (speedup 1.0000x reference)
import jax
import jax.numpy as jnp
from jax import lax
from jax.experimental import pallas as pl
from jax.experimental.pallas import tpu as pltpu
from jax.experimental.pallas import tpu_sc as plsc

N = 10000
E = 320000
NC, NS = 2, 16
CHUNK = 128
NCHT = 2500
NCH = 157
ACC_ROWS = 10240
ZROWS = ACC_ROWS // NS

_mesh = plsc.VectorSubcoreMesh(
    core_axis_name="c", subcore_axis_name="s", num_cores=NC, num_subcores=NS
)


def _make_sc_aggregate(D: int, B: int, LA: int):
  H = D // 2

  def body(support, edges, out, acc, isrc, idst, rows, gsem, ssem, isem):
    c = lax.axis_index("c")
    s = lax.axis_index("s")
    start = s * NCHT // NS
    nch = (s + 1) * NCHT // NS - start

    ist = pltpu.async_copy(edges.at[0].at[pl.ds(start, NCH)], isrc, isem)
    idt = pltpu.async_copy(edges.at[1].at[pl.ds(start, NCH)], idst, isem)

    tab = support.at[c]

    def g_start(m, b):
      return pltpu.async_copy(tab.at[isrc.at[m]], rows.at[b], gsem.at[b])

    def g_wait(m, b):
      pltpu.make_async_copy(tab.at[isrc.at[m]], rows.at[b], gsem.at[b]).wait()

    def s_start(m, b):
      return pltpu.async_copy(
          rows.at[b], acc.at[idst.at[m]], ssem.at[b], add=True)

    def s_wait(m, b):
      pltpu.make_async_copy(
          rows.at[b], acc.at[idst.at[m]], ssem.at[b]).wait()

    SL = B - LA
    NB = 156 // B
    ist.wait()
    idt.wait()
    for m in range(LA):
      g_start(m, m % B)

    zbuf = rows.at[B - 1]

    def zrow(i, _):
      for k in range(H // 16):
        zbuf[i, pl.ds(k * 16, 16)] = jnp.zeros((16,), jnp.float32)
      return 0

    lax.fori_loop(0, CHUNK, zrow, 0)
    zb = s * ZROWS
    off = 0
    while off < ZROWS:
      sz = min(CHUNK, ZROWS - off)
      pltpu.sync_copy(zbuf.at[pl.ds(0, sz)], acc.at[pl.ds(zb + off, sz)])
      off += sz
    plsc.subcore_barrier()


    for m in range(B):
      b = (m + LA) % B
      if m >= SL:
        s_wait(m - SL, b)
      g_start(m + LA, b)
      g_wait(m, m % B)
      s_start(m, m % B)

    def group(h, _):
      j = B * h
      for k in range(B):
        m = j + k
        b = (m + LA) % B
        s_wait(m - SL, b)
        g_start(m + LA, b)
        g_wait(m, m % B)
        s_start(m, m % B)
      return 0

    lax.fori_loop(1, NB - 1, group, 0)

    for k in range(B):
      m = 156 - B + k
      b = (m + LA) % B
      s_wait(m - SL, b)
      if m + LA < 156:
        g_start(m + LA, b)
      g_wait(m, m % B)
      s_start(m, m % B)
    for m in range(156 - SL, 156):
      s_wait(m, m % B)

    @pl.when(nch == NCH)
    def _():
      g_start(156, 0).wait()
      s_start(156, 0).wait()

    plsc.subcore_barrier()

    ob = s * ZROWS
    last = N - (NS - 1) * ZROWS

    @pl.when(s < NS - 1)
    def _():
      pltpu.sync_copy(acc.at[pl.ds(ob, ZROWS)],
                      out.at[pl.ds(ob, ZROWS), pl.ds(c * H, H)])

    @pl.when(s == NS - 1)
    def _():
      pltpu.sync_copy(acc.at[pl.ds(ob, last)],
                      out.at[pl.ds(ob, last), pl.ds(c * H, H)])

  return pl.kernel(
      body,
      out_type=jax.ShapeDtypeStruct((N, D), jnp.float32),
      mesh=_mesh,
      scratch_types=[
          pltpu.VMEM_SHARED((ACC_ROWS, H), jnp.float32),
          pltpu.VMEM((NCH, CHUNK), jnp.int32),
          pltpu.VMEM((NCH, CHUNK), jnp.int32),
          pltpu.VMEM((B, CHUNK, H), jnp.float32),
          pltpu.SemaphoreType.DMA((B,)),
          pltpu.SemaphoreType.DMA((B,)),
          pltpu.SemaphoreType.DMA,
      ],
      compiler_params=pltpu.CompilerParams(use_tc_tiling_on_sc=False),
  )


_sc_agg_128 = _make_sc_aggregate(128, B=6, LA=4)
_sc_agg_64 = _make_sc_aggregate(64, B=6, LA=4)


def _mm_bias_split_kernel(x_ref, w_ref, b_ref, o_ref):
  h = o_ref.shape[-1]
  d = (
      jnp.dot(x_ref[...], w_ref[...], preferred_element_type=jnp.float32)
      + b_ref[...]
  )
  o_ref[0] = d[:, :h]
  o_ref[1] = d[:, h:]


_BN = 2000


def _tc_mm_bias_split(x, w, b):
  din, dout = w.shape
  h = dout // 2
  return pl.pallas_call(
      _mm_bias_split_kernel,
      grid=(N // _BN,),
      in_specs=[
          pl.BlockSpec((_BN, din), lambda i: (i, 0)),
          pl.BlockSpec((din, dout), lambda i: (0, 0)),
          pl.BlockSpec((1, dout), lambda i: (0, 0)),
      ],
      out_specs=pl.BlockSpec((2, _BN, h), lambda i: (0, i, 0)),
      out_shape=jax.ShapeDtypeStruct((2, N, h), jnp.float32),
  )(x, w, b.reshape(1, dout))


@jax.jit
def kernel(fea, adj, W1, b1, W2, b2):
  edges = adj.astype(jnp.int32).reshape(2, NCHT, CHUNK)

  s1 = _tc_mm_bias_split(fea, W1, b1)
  x1 = _sc_agg_128(s1, edges)
  s2 = _tc_mm_bias_split(x1, W2, b2)
  return _sc_agg_64(s2, edges)


# speedup vs baseline: 1.0074x; 1.0074x over previous
import jax
import jax.numpy as jnp
from jax import lax
from jax.experimental import pallas as pl
from jax.experimental.pallas import tpu as pltpu
from jax.experimental.pallas import tpu_sc as plsc

N = 10000
E = 320000
NC, NS = 2, 16
CHUNK = 128
NCHT = 2500
NCH = 157
ACC_ROWS = 10240
ZROWS = ACC_ROWS // NS

_mesh = plsc.VectorSubcoreMesh(
    core_axis_name="c", subcore_axis_name="s", num_cores=NC, num_subcores=NS
)


def _make_sc_aggregate(D: int, B: int, LA: int):
  H = D // 2

  def body(support, edges, out, acc, isrc, idst, rows, gsem, ssem, isem):
    c = lax.axis_index("c")
    s = lax.axis_index("s")
    start = s * NCHT // NS
    nch = (s + 1) * NCHT // NS - start

    ist = pltpu.async_copy(edges.at[0].at[pl.ds(start, NCH)], isrc, isem)
    idt = pltpu.async_copy(edges.at[1].at[pl.ds(start, NCH)], idst, isem)

    tab = support.at[c]

    def g_start(m, b):
      return pltpu.async_copy(tab.at[isrc.at[m]], rows.at[b], gsem.at[b])

    def g_wait(m, b):
      pltpu.make_async_copy(tab.at[isrc.at[m]], rows.at[b], gsem.at[b]).wait()

    def s_start(m, b):
      return pltpu.async_copy(
          rows.at[b], acc.at[idst.at[m]], ssem.at[b], add=True)

    def s_wait(m, b):
      pltpu.make_async_copy(
          rows.at[b], acc.at[idst.at[m]], ssem.at[b]).wait()

    SL = B - LA
    NB = 156 // B
    ist.wait()
    idt.wait()
    for m in range(LA):
      g_start(m, m % B)

    zbuf = rows.at[B - 1]

    def zrow(i, _):
      for k in range(H // 16):
        zbuf[i, pl.ds(k * 16, 16)] = jnp.zeros((16,), jnp.float32)
      return 0

    lax.fori_loop(0, CHUNK, zrow, 0)
    zb = s * ZROWS
    off = 0
    while off < ZROWS:
      sz = min(CHUNK, ZROWS - off)
      pltpu.sync_copy(zbuf.at[pl.ds(0, sz)], acc.at[pl.ds(zb + off, sz)])
      off += sz
    plsc.subcore_barrier()


    for m in range(B):
      b = (m + LA) % B
      if m >= SL:
        s_wait(m - SL, b)
      g_start(m + LA, b)
      g_wait(m, m % B)
      s_start(m, m % B)

    def group(h, _):
      j = B * h
      for k in range(B):
        m = j + k
        b = (m + LA) % B
        s_wait(m - SL, b)
        g_start(m + LA, b)
        g_wait(m, m % B)
        s_start(m, m % B)
      return 0

    lax.fori_loop(1, NB - 1, group, 0)

    for k in range(B):
      m = 156 - B + k
      b = (m + LA) % B
      s_wait(m - SL, b)
      if m + LA < 156:
        g_start(m + LA, b)
      elif m + LA == 156:
        @pl.when(nch == NCH)
        def _():
          g_start(156, b)
      g_wait(m, m % B)
      s_start(m, m % B)
    for m in range(156 - SL, 156):
      s_wait(m, m % B)

    @pl.when(nch == NCH)
    def _():
      g_wait(156, 156 % B)
      s_start(156, 156 % B).wait()

    plsc.subcore_barrier()

    ob = s * ZROWS
    last = N - (NS - 1) * ZROWS

    @pl.when(s < NS - 1)
    def _():
      pltpu.sync_copy(acc.at[pl.ds(ob, ZROWS)],
                      out.at[pl.ds(ob, ZROWS), pl.ds(c * H, H)])

    @pl.when(s == NS - 1)
    def _():
      pltpu.sync_copy(acc.at[pl.ds(ob, last)],
                      out.at[pl.ds(ob, last), pl.ds(c * H, H)])

  return pl.kernel(
      body,
      out_type=jax.ShapeDtypeStruct((N, D), jnp.float32),
      mesh=_mesh,
      scratch_types=[
          pltpu.VMEM_SHARED((ACC_ROWS, H), jnp.float32),
          pltpu.VMEM((NCH, CHUNK), jnp.int32),
          pltpu.VMEM((NCH, CHUNK), jnp.int32),
          pltpu.VMEM((B, CHUNK, H), jnp.float32),
          pltpu.SemaphoreType.DMA((B,)),
          pltpu.SemaphoreType.DMA((B,)),
          pltpu.SemaphoreType.DMA,
      ],
      compiler_params=pltpu.CompilerParams(use_tc_tiling_on_sc=False),
  )


_sc_agg_128 = _make_sc_aggregate(128, B=6, LA=4)
_sc_agg_64 = _make_sc_aggregate(64, B=6, LA=4)


def _mm_bias_split_kernel(x_ref, w_ref, b_ref, o_ref):
  h = o_ref.shape[-1]
  d = (
      jnp.dot(x_ref[...], w_ref[...], preferred_element_type=jnp.float32)
      + b_ref[...]
  )
  o_ref[0] = d[:, :h]
  o_ref[1] = d[:, h:]


_BN = 2000


def _tc_mm_bias_split(x, w, b):
  din, dout = w.shape
  h = dout // 2
  return pl.pallas_call(
      _mm_bias_split_kernel,
      grid=(N // _BN,),
      in_specs=[
          pl.BlockSpec((_BN, din), lambda i: (i, 0)),
          pl.BlockSpec((din, dout), lambda i: (0, 0)),
          pl.BlockSpec((1, dout), lambda i: (0, 0)),
      ],
      out_specs=pl.BlockSpec((2, _BN, h), lambda i: (0, i, 0)),
      out_shape=jax.ShapeDtypeStruct((2, N, h), jnp.float32),
  )(x, w, b.reshape(1, dout))


@jax.jit
def kernel(fea, adj, W1, b1, W2, b2):
  edges = adj.astype(jnp.int32).reshape(2, NCHT, CHUNK)

  s1 = _tc_mm_bias_split(fea, W1, b1)
  x1 = _sc_agg_128(s1, edges)
  s2 = _tc_mm_bias_split(x1, W2, b2)
  return _sc_agg_64(s2, edges)
